# baseline (speedup 1.0000x reference)
import functools
import jax
import jax.numpy as jnp
from jax import lax
from jax.experimental import pallas as pl
from jax.experimental.pallas import tpu as pltpu
from jax.experimental.pallas import tpu_sc as plsc

D_MODEL_ = 768
N_EXP_ = 64
NC_, NS_, L_ = 2, 16, 16
NW_ = NC_ * NS_


def _matmul_body(x_ref, wt_ref, logits_ref):
    logits_ref[...] = jnp.dot(x_ref[...], wt_ref[...],
                              preferred_element_type=jnp.float32)


def _tc_logits(xf, wt, T):
    BT = 4096
    return pl.pallas_call(
        _matmul_body,
        grid=(T // BT,),
        in_specs=[
            pl.BlockSpec((BT, D_MODEL_), lambda i: (i, 0)),
            pl.BlockSpec((D_MODEL_, N_EXP_), lambda i: (0, 0)),
        ],
        out_specs=pl.BlockSpec((BT, N_EXP_), lambda i: (i, 0)),
        out_shape=jax.ShapeDtypeStruct((T, N_EXP_), jnp.float32),
    )(xf, wt)


def _sc_topk(logits_flat, T):
    per_w = T // NW_
    n_groups = per_w // L_

    mesh = plsc.VectorSubcoreMesh(core_axis_name="c", subcore_axis_name="s")

    @functools.partial(
        pl.kernel,
        mesh=mesh,
        out_type=[
            jax.ShapeDtypeStruct((T,), jnp.float32),
            jax.ShapeDtypeStruct((T,), jnp.float32),
            jax.ShapeDtypeStruct((T,), jnp.int32),
            jax.ShapeDtypeStruct((T,), jnp.int32),
        ],
        scratch_types=[
            pltpu.VMEM((per_w * N_EXP_,), jnp.float32),
            pltpu.VMEM((per_w,), jnp.float32),
            pltpu.VMEM((per_w,), jnp.float32),
            pltpu.VMEM((per_w,), jnp.int32),
            pltpu.VMEM((per_w,), jnp.int32),
        ],
        compiler_params=pltpu.CompilerParams(needs_layout_passes=False),
    )
    def topk_kernel(logits_hbm, w1_hbm, w2_hbm, i1_hbm, i2_hbm,
                    lv, w1v, w2v, i1v, i2v):
        wid = lax.axis_index("s") * NC_ + lax.axis_index("c")
        base = wid * per_w
        pltpu.sync_copy(logits_hbm.at[pl.ds(base * N_EXP_, per_w * N_EXP_)],
                        lv)

        iota16 = lax.iota(jnp.int32, L_)
        neg_inf = jnp.full((L_,), -jnp.inf, jnp.float32)
        zeros_i = jnp.zeros((L_,), jnp.int32)

        def group_body(g, _):
            idx0 = (g * L_ + iota16) * N_EXP_
            m1, m2, i1, i2 = neg_inf, neg_inf, zeros_i, zeros_i
            for e in range(N_EXP_):
                e_vec = jnp.full((L_,), e, jnp.int32)
                v = plsc.load_gather(lv, [idx0 + e])
                gt1 = v > m1
                gt2 = v > m2
                m2n = jnp.where(gt1, m1, jnp.where(gt2, v, m2))
                i2n = jnp.where(gt1, i1, jnp.where(gt2, e_vec, i2))
                m1 = jnp.where(gt1, v, m1)
                i1 = jnp.where(gt1, e_vec, i1)
                m2, i2 = m2n, i2n
            ex = jnp.exp(m2 - m1)
            denom = 1.0 + ex
            sl = pl.ds(g * L_, L_)
            w1v[sl] = 1.0 / denom
            w2v[sl] = ex / denom
            i1v[sl] = i1
            i2v[sl] = i2
            return ()

        lax.fori_loop(0, n_groups, group_body, ())

        pltpu.sync_copy(w1v, w1_hbm.at[pl.ds(base, per_w)])
        pltpu.sync_copy(w2v, w2_hbm.at[pl.ds(base, per_w)])
        pltpu.sync_copy(i1v, i1_hbm.at[pl.ds(base, per_w)])
        pltpu.sync_copy(i2v, i2_hbm.at[pl.ds(base, per_w)])

    return topk_kernel(logits_flat)


def kernel(x, W_gate):
    B, S, D = x.shape
    T = B * S
    xf = x.reshape(T, D)
    wt = W_gate.T

    logits = _tc_logits(xf, wt, T)
    w1, w2, i1, i2 = _sc_topk(logits.reshape(T * N_EXP_), T)

    weights = jnp.stack([w1, w2], axis=-1).reshape(B, S, 2)
    indices = jnp.stack([i1, i2], axis=-1).reshape(B, S, 2)
    return (weights, indices, logits.reshape(B, S, N_EXP_))
